# baseline (speedup 1.0000x reference)
import jax
import jax.numpy as jnp
from jax.experimental import pallas as pl
from jax.experimental.pallas import tpu as pltpu

_MEDNUM = 2048
_FDIM = 64
_BR = 256


def _ddi_gcn_kernel(adj_ref, w_ref, inter_ref, out_ref):
    w = w_ref[:]
    a1 = adj_ref[0, :, :_MEDNUM] + adj_ref[0, :, _MEDNUM:]
    a2 = adj_ref[1, :, :_MEDNUM] + adj_ref[1, :, _MEDNUM:]
    y1 = jnp.maximum(jnp.dot(a1, w, preferred_element_type=jnp.float32), 0.0)
    y2 = jnp.maximum(jnp.dot(a2, w, preferred_element_type=jnp.float32), 0.0)
    t = inter_ref[0, 0]
    out_ref[:] = (2.0 * t) * y1 + (2.0 - 2.0 * t) * y2


@jax.jit
def kernel(adj1, mEmbed, inter):
    n_tiles = _MEDNUM // _BR
    adj3 = adj1.reshape(2, _MEDNUM, 2 * _MEDNUM)
    return pl.pallas_call(
        _ddi_gcn_kernel,
        grid=(n_tiles,),
        in_specs=[
            pl.BlockSpec((2, _BR, 2 * _MEDNUM), lambda j: (0, j, 0)),
            pl.BlockSpec((_MEDNUM, _FDIM), lambda j: (0, 0)),
            pl.BlockSpec((1, 1), lambda j: (0, 0)),
        ],
        out_specs=pl.BlockSpec((_BR, _FDIM), lambda j: (j, 0)),
        out_shape=jax.ShapeDtypeStruct((_MEDNUM, _FDIM), jnp.float32),
        compiler_params=pltpu.CompilerParams(
            vmem_limit_bytes=112 * 1024 * 1024,
            skip_device_barrier=True,
            disable_bounds_checks=True,
            disable_semaphore_checks=True,
        ),
    )(adj3, mEmbed, inter.reshape(1, 1))


# speedup vs baseline: 1.2402x; 1.2402x over previous
import jax
import jax.numpy as jnp
from jax.experimental import pallas as pl
from jax.experimental.pallas import tpu as pltpu

_MEDNUM = 2048
_FDIM = 64
_BR = 256

_DN = (((1,), (1,)), ((), ()))


def _ddi_gcn_kernel(adj_ref, wt_ref, inter_ref, out_ref):
    wt = wt_ref[:]
    a1 = adj_ref[0, :, :_MEDNUM] + adj_ref[0, :, _MEDNUM:]
    a2 = adj_ref[1, :, :_MEDNUM] + adj_ref[1, :, _MEDNUM:]
    yt1 = jax.lax.dot_general(wt, a1, _DN, preferred_element_type=jnp.float32)
    yt2 = jax.lax.dot_general(wt, a2, _DN, preferred_element_type=jnp.float32)
    yt1 = jnp.maximum(yt1, 0.0)
    yt2 = jnp.maximum(yt2, 0.0)
    t = inter_ref[0, 0]
    out_ref[:] = (2.0 * t) * yt1 + (2.0 - 2.0 * t) * yt2


@jax.jit
def kernel(adj1, mEmbed, inter):
    n_tiles = _MEDNUM // _BR
    adj3 = adj1.reshape(2, _MEDNUM, 2 * _MEDNUM)
    out_t = pl.pallas_call(
        _ddi_gcn_kernel,
        grid=(n_tiles,),
        in_specs=[
            pl.BlockSpec((2, _BR, 2 * _MEDNUM), lambda j: (0, j, 0)),
            pl.BlockSpec((_FDIM, _MEDNUM), lambda j: (0, 0)),
            pl.BlockSpec((1, 1), lambda j: (0, 0)),
        ],
        out_specs=pl.BlockSpec((_FDIM, _BR), lambda j: (0, j)),
        out_shape=jax.ShapeDtypeStruct((_FDIM, _MEDNUM), jnp.float32),
        compiler_params=pltpu.CompilerParams(
            vmem_limit_bytes=112 * 1024 * 1024
        ),
    )(adj3, mEmbed.T, inter.reshape(1, 1))
    return out_t.T
